# baseline (speedup 1.0000x reference)
import functools

import jax
import jax.numpy as jnp
from jax import lax
from jax.experimental import pallas as pl
from jax.experimental.pallas import tpu as pltpu
from jax.experimental.pallas import tpu_sc as plsc

N = 100000
NH = N // 2
C = 128
S = 12500
NW = 32
SEG_PAD = 12544
SB = SEG_PAD // NW
RB = 2000
CH = 256
L = 16


def _proj_body(seg_ref, feats_ref, w_ref, b_ref, x_ref, counts_ref):
    i = pl.program_id(0)
    xb = jnp.dot(feats_ref[...], w_ref[...], preferred_element_type=jnp.float32)
    x_ref[...] = xb + b_ref[...]
    seg = seg_ref[0, 0, :]
    thr = SB * lax.broadcasted_iota(jnp.int32, (1, 128), 1)
    cmp = (seg[:, None] < thr).astype(jnp.int32)
    csum = jnp.sum(cmp, axis=0, keepdims=True)

    @pl.when(i == 0)
    def _():
        counts_ref[...] = jnp.zeros((8, 128), jnp.int32)

    counts_ref[...] += jnp.broadcast_to(csum, (8, 128))


def _project(feats_half, seg3d_half, W, b2d):
    return pl.pallas_call(
        _proj_body,
        grid=(NH // RB,),
        in_specs=[
            pl.BlockSpec((1, 1, RB), lambda i: (i, 0, 0)),
            pl.BlockSpec((RB, C), lambda i: (i, 0)),
            pl.BlockSpec((C, C), lambda i: (0, 0)),
            pl.BlockSpec((1, C), lambda i: (0, 0)),
        ],
        out_specs=[
            pl.BlockSpec((RB, C), lambda i: (i, 0)),
            pl.BlockSpec((8, 128), lambda i: (0, 0)),
        ],
        out_shape=[
            jax.ShapeDtypeStruct((NH + CH, C), jnp.float32),
            jax.ShapeDtypeStruct((8, 128), jnp.int32),
        ],
    )(seg3d_half, feats_half, W, b2d)


def _sread(stv, idx):
    return stv[pl.ds(idx, L)][0]


def _seg_max_half(x, seg_pad, starts):
    mesh = plsc.VectorSubcoreMesh(core_axis_name="c", subcore_axis_name="s")

    @functools.partial(
        pl.kernel,
        mesh=mesh,
        compiler_params=pltpu.CompilerParams(needs_layout_passes=False),
        out_type=jax.ShapeDtypeStruct((SEG_PAD, C), jnp.float32),
        scratch_types=[
            pltpu.VMEM((SB + 8, C), jnp.float32),
            pltpu.VMEM((CH, C), jnp.float32),
            pltpu.VMEM((CH, C), jnp.float32),
            pltpu.VMEM((CH,), jnp.int32),
            pltpu.VMEM((CH,), jnp.int32),
            pltpu.VMEM((128,), jnp.int32),
            pltpu.SemaphoreType.DMA,
            pltpu.SemaphoreType.DMA,
        ],
    )
    def body(x_hbm, seg_hbm, starts_hbm, out_hbm,
             loc, xch0, xch1, segch0, segch1, stv, sem0, sem1):
        wid = lax.axis_index("c") * 16 + lax.axis_index("s")
        seg_base = pl.multiple_of(wid * SB, 8)
        pltpu.sync_copy(starts_hbm, stv)
        rs = _sread(stv, wid)
        re = _sread(stv, wid + 1)
        rb0 = (rs // 8) * 8
        nchunks = (re - rb0 + CH - 1) // CH

        neg_inf = jnp.full((L,), -jnp.inf, jnp.float32)

        def init_row(i, carry):
            for g in range(8):
                loc[i, pl.ds(g * L, L)] = neg_inf
            return carry

        lax.fori_loop(0, SB, init_row, 0)

        seg_base_v = jnp.full((L,), seg_base, jnp.int32)
        dump_v = jnp.full((L,), SB, jnp.int32)

        def chunk_slices(ci):
            rb = pl.multiple_of(jnp.minimum(rb0 + ci * CH, NH), 8)
            return (x_hbm.at[pl.ds(rb, CH)],
                    seg_hbm.at[pl.ds(rb, CH)])

        def start_fetch(ci, xb, sb, sem):
            xs, ss = chunk_slices(ci)
            pltpu.async_copy(xs, xb, sem)
            pltpu.async_copy(ss, sb, sem)

        def wait_fetch(ci, xb, sb, sem):
            xs, ss = chunk_slices(ci)
            pltpu.make_async_copy(xs, xb, sem).wait()
            pltpu.make_async_copy(ss, sb, sem).wait()

        def make_block(xb, sb):
            def do_block(r16, carry):
                runs, prev = carry
                segv = sb[pl.ds(r16 * L, L)]
                slv = segv - seg_base_v
                valid = (slv >= 0) & (slv < SB)
                sd = jnp.where(valid, slv, dump_v)
                for r in range(L):
                    s = sd[r]
                    change = s != prev
                    ridx = r16 * L + r

                    @pl.when(change)
                    def _(runs=runs, prev=prev):
                        for g in range(8):
                            loc[prev, pl.ds(g * L, L)] = runs[g]

                    new_runs = []
                    for g in range(8):
                        xg = xb[ridx, pl.ds(g * L, L)]
                        rg = jnp.where(change, xg, jnp.maximum(runs[g], xg))
                        new_runs.append(rg)
                    runs = tuple(new_runs)
                    prev = s
                return runs, prev
            return do_block

        block0 = make_block(xch0, segch0)
        block1 = make_block(xch1, segch1)

        def pair_body(p, carry):
            i0 = 2 * p
            start_fetch(i0 + 1, xch1, segch1, sem1)
            wait_fetch(i0, xch0, segch0, sem0)
            carry = lax.fori_loop(0, CH // L, block0, carry)
            start_fetch(i0 + 2, xch0, segch0, sem0)
            wait_fetch(i0 + 1, xch1, segch1, sem1)
            carry = lax.fori_loop(0, CH // L, block1, carry)
            return carry

        runs0 = tuple(jnp.full((L,), -jnp.inf, jnp.float32) for _ in range(8))
        prev0 = jnp.full((), SB, jnp.int32)
        start_fetch(0, xch0, segch0, sem0)
        npairs = (nchunks + 1) // 2
        runsf, prevf = lax.fori_loop(0, npairs, pair_body, (runs0, prev0))
        for g in range(8):
            loc[prevf, pl.ds(g * L, L)] = runsf[g]
        wait_fetch(2 * npairs, xch0, segch0, sem0)

        pltpu.sync_copy(loc.at[pl.ds(0, SB)], out_hbm.at[pl.ds(seg_base, SB)])

    return body(x, seg_pad, starts)


def _ln_gelu_body(pa_ref, pb_ref, g_ref, be_ref, o_ref):
    p = jnp.maximum(pa_ref[...], pb_ref[...])
    p = jnp.where(jnp.isfinite(p), p, 0.0)
    mean = jnp.mean(p, axis=-1, keepdims=True)
    var = jnp.mean((p - mean) ** 2, axis=-1, keepdims=True)
    y = (p - mean) * lax.rsqrt(var + 1e-5) * g_ref[...] + be_ref[...]
    o_ref[...] = jax.nn.gelu(y)


def _ln_gelu(pa, pb, gamma2d, beta2d):
    blk = 1568
    return pl.pallas_call(
        _ln_gelu_body,
        grid=(SEG_PAD // blk,),
        in_specs=[
            pl.BlockSpec((blk, C), lambda i: (i, 0)),
            pl.BlockSpec((blk, C), lambda i: (i, 0)),
            pl.BlockSpec((1, C), lambda i: (0, 0)),
            pl.BlockSpec((1, C), lambda i: (0, 0)),
        ],
        out_specs=pl.BlockSpec((blk, C), lambda i: (i, 0)),
        out_shape=jax.ShapeDtypeStruct((S, C), jnp.float32),
    )(pa, pb, gamma2d, beta2d)


def kernel(feats, segment_ids, W, b, gamma, beta):
    b2d = b.reshape(1, C)
    sent = jnp.full((CH,), 2**30, jnp.int32)
    seg_a = segment_ids[:NH]
    seg_b = segment_ids[NH:]
    xa, counts_a = _project(feats[:NH], seg_a.reshape(NH // RB, 1, RB), W, b2d)
    pooled_a = _seg_max_half(
        xa, jnp.concatenate([seg_a, sent]), counts_a[0])
    xb, counts_b = _project(feats[NH:], seg_b.reshape(NH // RB, 1, RB), W, b2d)
    pooled_b = _seg_max_half(
        xb, jnp.concatenate([seg_b, sent]), counts_b[0])
    return _ln_gelu(pooled_a, pooled_b, gamma.reshape(1, C), beta.reshape(1, C))


# speedup vs baseline: 1.4333x; 1.4333x over previous
import functools

import jax
import jax.numpy as jnp
from jax import lax
from jax.experimental import pallas as pl
from jax.experimental.pallas import tpu as pltpu
from jax.experimental.pallas import tpu_sc as plsc

N = 100000
C = 128
S = 12500
NW = 32
SEG_PAD = 12544
SB = SEG_PAD // NW
RB = 2000
CH = 256
L = 16


def _proj_body(seg_ref, feats_ref, w_ref, b_ref, x_ref, counts_ref):
    i = pl.program_id(0)
    xb = jnp.dot(feats_ref[...], w_ref[...], preferred_element_type=jnp.float32)
    x_ref[...] = xb + b_ref[...]
    seg = seg_ref[0, 0, :]
    thr = SB * lax.broadcasted_iota(jnp.int32, (1, 128), 1)
    cmp = (seg[:, None] < thr).astype(jnp.int32)
    csum = jnp.sum(cmp, axis=0, keepdims=True)

    @pl.when(i == 0)
    def _():
        counts_ref[...] = jnp.zeros((8, 128), jnp.int32)

    counts_ref[...] += jnp.broadcast_to(csum, (8, 128))


def _project(feats, seg3d, W, b2d):
    return pl.pallas_call(
        _proj_body,
        grid=(N // RB,),
        in_specs=[
            pl.BlockSpec((1, 1, RB), lambda i: (i, 0, 0)),
            pl.BlockSpec((RB, C), lambda i: (i, 0)),
            pl.BlockSpec((C, C), lambda i: (0, 0)),
            pl.BlockSpec((1, C), lambda i: (0, 0)),
        ],
        out_specs=[
            pl.BlockSpec((RB, C), lambda i: (i, 0)),
            pl.BlockSpec((8, 128), lambda i: (0, 0)),
        ],
        out_shape=[
            jax.ShapeDtypeStruct((N + CH, C), jnp.float32),
            jax.ShapeDtypeStruct((8, 128), jnp.int32),
        ],
    )(seg3d, feats, W, b2d)


def _sread(stv, idx):
    return stv[pl.ds(idx, L)][0]


def _seg_max(x, seg_pad, starts):
    mesh = plsc.VectorSubcoreMesh(core_axis_name="c", subcore_axis_name="s")

    @functools.partial(
        pl.kernel,
        mesh=mesh,
        compiler_params=pltpu.CompilerParams(needs_layout_passes=False),
        out_type=jax.ShapeDtypeStruct((SEG_PAD, C), jnp.float32),
        scratch_types=[
            pltpu.VMEM((SB + 8, C), jnp.float32),
            pltpu.VMEM((CH, C), jnp.float32),
            pltpu.VMEM((CH, C), jnp.float32),
            pltpu.VMEM((CH,), jnp.int32),
            pltpu.VMEM((CH,), jnp.int32),
            pltpu.VMEM((128,), jnp.int32),
            pltpu.SemaphoreType.DMA,
            pltpu.SemaphoreType.DMA,
        ],
    )
    def body(x_hbm, seg_hbm, starts_hbm, out_hbm,
             loc, xch0, xch1, segch0, segch1, stv, sem0, sem1):
        wid = lax.axis_index("c") * 16 + lax.axis_index("s")
        seg_base = pl.multiple_of(wid * SB, 8)
        pltpu.sync_copy(starts_hbm, stv)
        rs = _sread(stv, wid)
        re = _sread(stv, wid + 1)
        rb0 = (rs // 8) * 8
        nchunks = (re - rb0 + CH - 1) // CH

        neg_inf = jnp.full((L,), -jnp.inf, jnp.float32)

        def init_row(i, carry):
            for g in range(8):
                loc[i, pl.ds(g * L, L)] = neg_inf
            return carry

        lax.fori_loop(0, SB, init_row, 0)

        lanes = lax.iota(jnp.int32, L)
        gdn = lax.GatherDimensionNumbers(
            offset_dims=(), collapsed_slice_dims=(0,), start_index_map=(0,))

        def bcast_lane(vec, r):
            idxr = jnp.full((L, 1), r, jnp.int32)
            return lax.gather(vec, idxr, gdn, slice_sizes=(1,),
                              mode=lax.GatherScatterMode.PROMISE_IN_BOUNDS)

        seg_base_v = jnp.full((L,), seg_base, jnp.int32)
        dump_v = jnp.full((L,), SB, jnp.int32)
        colidx = [g * L + lanes for g in range(8)]

        def chunk_slices(ci):
            rb = pl.multiple_of(jnp.minimum(rb0 + ci * CH, N), 8)
            return (x_hbm.at[pl.ds(rb, CH)],
                    seg_hbm.at[pl.ds(rb, CH)])

        def start_fetch(ci, xb, sb, sem):
            xs, ss = chunk_slices(ci)
            pltpu.async_copy(xs, xb, sem)
            pltpu.async_copy(ss, sb, sem)

        def wait_fetch(ci, xb, sb, sem):
            xs, ss = chunk_slices(ci)
            pltpu.make_async_copy(xs, xb, sem).wait()
            pltpu.make_async_copy(ss, sb, sem).wait()

        def make_block(xb, sb):
            def do_block(r16, carry):
                runs, prev = carry
                segv = sb[pl.ds(r16 * L, L)]
                slv = segv - seg_base_v
                valid = (slv >= 0) & (slv < SB)
                sd = jnp.where(valid, slv, dump_v)
                for r in range(L):
                    s = sd[r]
                    change = s != prev
                    ridx = r16 * L + r

                    @pl.when(change)
                    def _(runs=runs, prev=prev):
                        for g in range(8):
                            loc[prev, pl.ds(g * L, L)] = runs[g]

                    new_runs = []
                    for g in range(8):
                        xg = xb[ridx, pl.ds(g * L, L)]
                        rg = jnp.where(change, xg, jnp.maximum(runs[g], xg))
                        new_runs.append(rg)
                    runs = tuple(new_runs)
                    prev = s
                return runs, prev
            return do_block

        block0 = make_block(xch0, segch0)
        block1 = make_block(xch1, segch1)

        def pair_body(p, carry):
            i0 = 2 * p
            start_fetch(i0 + 1, xch1, segch1, sem1)
            wait_fetch(i0, xch0, segch0, sem0)
            carry = lax.fori_loop(0, CH // L, block0, carry)
            start_fetch(i0 + 2, xch0, segch0, sem0)
            wait_fetch(i0 + 1, xch1, segch1, sem1)
            carry = lax.fori_loop(0, CH // L, block1, carry)
            return carry

        runs0 = tuple(jnp.full((L,), -jnp.inf, jnp.float32) for _ in range(8))
        prev0 = jnp.full((), SB, jnp.int32)
        start_fetch(0, xch0, segch0, sem0)
        npairs = (nchunks + 1) // 2
        runsf, prevf = lax.fori_loop(0, npairs, pair_body, (runs0, prev0))
        for g in range(8):
            loc[prevf, pl.ds(g * L, L)] = runsf[g]
        wait_fetch(2 * npairs, xch0, segch0, sem0)

        pltpu.sync_copy(loc.at[pl.ds(0, SB)], out_hbm.at[pl.ds(seg_base, SB)])

    return body(x, seg_pad, starts)


def _ln_gelu_body(p_ref, g_ref, be_ref, o_ref):
    p = p_ref[...]
    p = jnp.where(jnp.isfinite(p), p, 0.0)
    mean = jnp.mean(p, axis=-1, keepdims=True)
    var = jnp.mean((p - mean) ** 2, axis=-1, keepdims=True)
    y = (p - mean) * lax.rsqrt(var + 1e-5) * g_ref[...] + be_ref[...]
    o_ref[...] = jax.nn.gelu(y)


def _ln_gelu(pooled, gamma2d, beta2d):
    blk = 1568
    return pl.pallas_call(
        _ln_gelu_body,
        grid=(SEG_PAD // blk,),
        in_specs=[
            pl.BlockSpec((blk, C), lambda i: (i, 0)),
            pl.BlockSpec((1, C), lambda i: (0, 0)),
            pl.BlockSpec((1, C), lambda i: (0, 0)),
        ],
        out_specs=pl.BlockSpec((blk, C), lambda i: (i, 0)),
        out_shape=jax.ShapeDtypeStruct((S, C), jnp.float32),
    )(pooled, gamma2d, beta2d)


def kernel(feats, segment_ids, W, b, gamma, beta):
    seg3d = segment_ids.reshape(N // RB, 1, RB)
    x, counts = _project(feats, seg3d, W, b.reshape(1, C))
    starts = counts[0]
    seg_pad = jnp.concatenate(
        [segment_ids, jnp.full((CH,), 2**30, jnp.int32)])
    pooled = _seg_max(x, seg_pad, starts)
    return _ln_gelu(pooled, gamma.reshape(1, C), beta.reshape(1, C))


# speedup vs baseline: 1.4759x; 1.0298x over previous
import functools

import jax
import jax.numpy as jnp
from jax import lax
from jax.experimental import pallas as pl
from jax.experimental.pallas import tpu as pltpu
from jax.experimental.pallas import tpu_sc as plsc

N = 100000
C = 128
S = 12500
NW = 32
SEG_PAD = 12544
SB = SEG_PAD // NW
RB = 2000
CH = 288
L = 16


def _proj_body(seg_ref, feats_ref, w_ref, b_ref, x_ref, counts_ref):
    i = pl.program_id(0)
    xb = jnp.dot(feats_ref[...], w_ref[...], preferred_element_type=jnp.float32)
    x_ref[...] = xb + b_ref[...]
    seg = seg_ref[0, 0, :]
    thr = SB * lax.broadcasted_iota(jnp.int32, (1, 128), 1)
    cmp = (seg[:, None] < thr).astype(jnp.int32)
    csum = jnp.sum(cmp, axis=0, keepdims=True)

    @pl.when(i == 0)
    def _():
        counts_ref[...] = jnp.zeros((8, 128), jnp.int32)

    counts_ref[...] += jnp.broadcast_to(csum, (8, 128))


def _project(feats, seg3d, W, b2d):
    return pl.pallas_call(
        _proj_body,
        grid=(N // RB,),
        in_specs=[
            pl.BlockSpec((1, 1, RB), lambda i: (i, 0, 0)),
            pl.BlockSpec((RB, C), lambda i: (i, 0)),
            pl.BlockSpec((C, C), lambda i: (0, 0)),
            pl.BlockSpec((1, C), lambda i: (0, 0)),
        ],
        out_specs=[
            pl.BlockSpec((RB, C), lambda i: (i, 0)),
            pl.BlockSpec((8, 128), lambda i: (0, 0)),
        ],
        out_shape=[
            jax.ShapeDtypeStruct((N + CH, C), jnp.float32),
            jax.ShapeDtypeStruct((8, 128), jnp.int32),
        ],
    )(seg3d, feats, W, b2d)


def _sread(stv, idx):
    return stv[pl.ds(idx, L)][0]


def _seg_max(x, seg_pad, starts):
    mesh = plsc.VectorSubcoreMesh(core_axis_name="c", subcore_axis_name="s")

    @functools.partial(
        pl.kernel,
        mesh=mesh,
        compiler_params=pltpu.CompilerParams(needs_layout_passes=False),
        out_type=jax.ShapeDtypeStruct((SEG_PAD, C), jnp.float32),
        scratch_types=[
            pltpu.VMEM((SB + 8, C), jnp.float32),
            pltpu.VMEM((CH, C), jnp.float32),
            pltpu.VMEM((CH, C), jnp.float32),
            pltpu.VMEM((CH,), jnp.int32),
            pltpu.VMEM((CH,), jnp.int32),
            pltpu.VMEM((128,), jnp.int32),
            pltpu.SemaphoreType.DMA,
            pltpu.SemaphoreType.DMA,
        ],
    )
    def body(x_hbm, seg_hbm, starts_hbm, out_hbm,
             loc, xch0, xch1, segch0, segch1, stv, sem0, sem1):
        wid = lax.axis_index("c") * 16 + lax.axis_index("s")
        seg_base = pl.multiple_of(wid * SB, 8)
        pltpu.sync_copy(starts_hbm, stv)
        rs = _sread(stv, wid)
        re = _sread(stv, wid + 1)
        rb0 = (rs // 8) * 8
        nchunks = (re - rb0 + CH - 1) // CH

        neg_inf = jnp.full((L,), -jnp.inf, jnp.float32)

        def init_row(i, carry):
            for g in range(8):
                loc[i, pl.ds(g * L, L)] = neg_inf
            return carry

        lanes = lax.iota(jnp.int32, L)
        gdn = lax.GatherDimensionNumbers(
            offset_dims=(), collapsed_slice_dims=(0,), start_index_map=(0,))

        def bcast_lane(vec, r):
            idxr = jnp.full((L, 1), r, jnp.int32)
            return lax.gather(vec, idxr, gdn, slice_sizes=(1,),
                              mode=lax.GatherScatterMode.PROMISE_IN_BOUNDS)

        seg_base_v = jnp.full((L,), seg_base, jnp.int32)
        dump_v = jnp.full((L,), SB, jnp.int32)
        colidx = [g * L + lanes for g in range(8)]

        def chunk_slices(ci):
            rb = pl.multiple_of(jnp.minimum(rb0 + ci * CH, N), 8)
            return (x_hbm.at[pl.ds(rb, CH)],
                    seg_hbm.at[pl.ds(rb, CH)])

        def start_fetch(ci, xb, sb, sem):
            xs, ss = chunk_slices(ci)
            pltpu.async_copy(xs, xb, sem)
            pltpu.async_copy(ss, sb, sem)

        def wait_fetch(ci, xb, sb, sem):
            xs, ss = chunk_slices(ci)
            pltpu.make_async_copy(xs, xb, sem).wait()
            pltpu.make_async_copy(ss, sb, sem).wait()

        def make_block(xb, sb):
            def do_block(r16, carry):
                runs, prev = carry
                segv = sb[pl.ds(r16 * L, L)]
                slv = segv - seg_base_v
                valid = (slv >= 0) & (slv < SB)
                sd = jnp.where(valid, slv, dump_v)
                for r in range(L):
                    s = sd[r]
                    change = s != prev
                    ridx = r16 * L + r

                    @pl.when(change)
                    def _(runs=runs, prev=prev):
                        for g in range(8):
                            loc[prev, pl.ds(g * L, L)] = runs[g]

                    new_runs = []
                    for g in range(8):
                        xg = xb[ridx, pl.ds(g * L, L)]
                        rg = jnp.where(change, xg, jnp.maximum(runs[g], xg))
                        new_runs.append(rg)
                    runs = tuple(new_runs)
                    prev = s
                return runs, prev
            return do_block

        block0 = make_block(xch0, segch0)
        block1 = make_block(xch1, segch1)

        def pair_body(p, carry):
            i0 = 2 * p
            start_fetch(i0 + 1, xch1, segch1, sem1)
            wait_fetch(i0, xch0, segch0, sem0)
            carry = lax.fori_loop(0, CH // L, block0, carry)
            start_fetch(i0 + 2, xch0, segch0, sem0)
            wait_fetch(i0 + 1, xch1, segch1, sem1)
            carry = lax.fori_loop(0, CH // L, block1, carry)
            return carry

        runs0 = tuple(jnp.full((L,), -jnp.inf, jnp.float32) for _ in range(8))
        prev0 = jnp.full((), SB, jnp.int32)
        start_fetch(0, xch0, segch0, sem0)
        lax.fori_loop(0, SB, init_row, 0)
        npairs = (nchunks + 1) // 2
        runsf, prevf = lax.fori_loop(0, npairs, pair_body, (runs0, prev0))
        for g in range(8):
            loc[prevf, pl.ds(g * L, L)] = runsf[g]
        wait_fetch(2 * npairs, xch0, segch0, sem0)

        pltpu.sync_copy(loc.at[pl.ds(0, SB)], out_hbm.at[pl.ds(seg_base, SB)])

    return body(x, seg_pad, starts)


def _ln_gelu_body(p_ref, g_ref, be_ref, o_ref):
    p = p_ref[...]
    p = jnp.where(jnp.isfinite(p), p, 0.0)
    mean = jnp.mean(p, axis=-1, keepdims=True)
    var = jnp.mean((p - mean) ** 2, axis=-1, keepdims=True)
    y = (p - mean) * lax.rsqrt(var + 1e-5) * g_ref[...] + be_ref[...]
    o_ref[...] = jax.nn.gelu(y)


def _ln_gelu(pooled, gamma2d, beta2d):
    blk = 1568
    return pl.pallas_call(
        _ln_gelu_body,
        grid=(SEG_PAD // blk,),
        in_specs=[
            pl.BlockSpec((blk, C), lambda i: (i, 0)),
            pl.BlockSpec((1, C), lambda i: (0, 0)),
            pl.BlockSpec((1, C), lambda i: (0, 0)),
        ],
        out_specs=pl.BlockSpec((blk, C), lambda i: (i, 0)),
        out_shape=jax.ShapeDtypeStruct((S, C), jnp.float32),
    )(pooled, gamma2d, beta2d)


def kernel(feats, segment_ids, W, b, gamma, beta):
    seg3d = segment_ids.reshape(N // RB, 1, RB)
    x, counts = _project(feats, seg3d, W, b.reshape(1, C))
    starts = counts[0]
    seg_pad = jnp.concatenate(
        [segment_ids, jnp.full((CH,), 2**30, jnp.int32)])
    pooled = _seg_max(x, seg_pad, starts)
    return _ln_gelu(pooled, gamma.reshape(1, C), beta.reshape(1, C))
